# baseline (speedup 1.0000x reference)
import jax
import jax.numpy as jnp
from jax.experimental import pallas as pl
from jax.experimental.pallas import tpu as pltpu

_N, _F, _H, _C = 100000, 128, 128, 64
_NP = _N // 2
_R = 1000
_S = _NP // _R
_NBUF = 10
_GROUPS = _S // _NBUF


def _mlp_kernel(x_hbm, w1_ref, b1_ref, w2s_ref, b2s_ref, out_hbm, *scratch):
    xbufs = scratch[:_NBUF]
    obufs = scratch[_NBUF:2 * _NBUF]
    in_sem = scratch[2 * _NBUF]
    out_sem = scratch[2 * _NBUF + 1]

    def in_copy(c, k):
        return pltpu.make_async_copy(
            x_hbm.at[pl.ds(c * _R, _R)], xbufs[k], in_sem.at[k])

    def out_copy(c, k):
        return pltpu.make_async_copy(
            obufs[k], out_hbm.at[pl.ds(c * _R, _R)], out_sem.at[k])

    for k in range(_NBUF):
        in_copy(k, k).start()

    w1 = w1_ref[...]
    b1 = b1_ref[...]
    w2s = w2s_ref[...]
    b2s = b2s_ref[...]

    def group(i, carry):
        for k in range(_NBUF):
            c = i * _NBUF + k
            in_copy(c, k).wait()

            @pl.when(i >= 1)
            def _():
                out_copy(c - _NBUF, k).wait()

            xv = xbufs[k][...]
            he = jax.lax.dot_general(
                xv[:, :_F], w1,
                dimension_numbers=(((1,), (1,)), ((), ())),
                preferred_element_type=jnp.float32)
            ho = jax.lax.dot_general(
                xv[:, _F:], w1,
                dimension_numbers=(((1,), (1,)), ((), ())),
                preferred_element_type=jnp.float32)
            h2 = jnp.concatenate(
                [jnp.maximum(he + b1, 0.0), jnp.maximum(ho + b1, 0.0)],
                axis=1)
            obufs[k][...] = jax.lax.dot_general(
                h2, w2s,
                dimension_numbers=(((1,), (0,)), ((), ())),
                preferred_element_type=jnp.float32) + b2s

            out_copy(c, k).start()

            @pl.when(c + _NBUF < _S)
            def _():
                in_copy(c + _NBUF, k).start()
        return carry

    jax.lax.fori_loop(0, _GROUPS, group, 0)

    for k in range(_NBUF):
        out_copy(_S - _NBUF + k, k).wait()


def kernel(nodeblocks, x, W_in, b_in, W_out, b_out):
    x2 = x.reshape(_NP, 2 * _F)
    b1 = b_in.reshape(1, _H)
    w2s = jnp.zeros((2 * _H, 2 * _C), jnp.float32)
    w2s = w2s.at[:_H, :_C].set(W_out.T).at[_H:, _C:].set(W_out.T)
    b2s = jnp.concatenate([b_out, b_out]).reshape(1, 2 * _C)
    scratch = (
        [pltpu.VMEM((_R, 2 * _F), jnp.float32) for _ in range(_NBUF)]
        + [pltpu.VMEM((_R, 2 * _C), jnp.float32) for _ in range(_NBUF)]
        + [pltpu.SemaphoreType.DMA((_NBUF,)),
           pltpu.SemaphoreType.DMA((_NBUF,))]
    )
    out2 = pl.pallas_call(
        _mlp_kernel,
        in_specs=[
            pl.BlockSpec(memory_space=pltpu.MemorySpace.HBM),
            pl.BlockSpec(memory_space=pltpu.MemorySpace.VMEM),
            pl.BlockSpec(memory_space=pltpu.MemorySpace.VMEM),
            pl.BlockSpec(memory_space=pltpu.MemorySpace.VMEM),
            pl.BlockSpec(memory_space=pltpu.MemorySpace.VMEM),
        ],
        out_specs=pl.BlockSpec(memory_space=pltpu.MemorySpace.HBM),
        out_shape=jax.ShapeDtypeStruct((_NP, 2 * _C), jnp.float32),
        scratch_shapes=scratch,
    )(x2, W_in, b1, w2s, b2s)
    return out2.reshape(_N, _C)


# speedup vs baseline: 1.4270x; 1.4270x over previous
import jax
import jax.numpy as jnp
from jax.experimental import pallas as pl
from jax.experimental.pallas import tpu as pltpu

_N, _F, _H, _C = 100000, 128, 128, 64
_R = 2000
_S = _N // _R
_NBUF = 10
_GROUPS = _S // _NBUF


def _mlp_kernel(x_hbm, w1_ref, b1_ref, w2a_ref, w2b_ref, b2_ref, out_hbm,
                *scratch):
    xbufs = scratch[:_NBUF]
    obufs = scratch[_NBUF:2 * _NBUF]
    in_sem = scratch[2 * _NBUF]
    out_sem = scratch[2 * _NBUF + 1]

    def in_copy(c, k):
        return pltpu.make_async_copy(
            x_hbm.at[pl.ds(c * _R, _R)], xbufs[k], in_sem.at[k])

    def out_copy(c, k):
        return pltpu.make_async_copy(
            obufs[k], out_hbm.at[pl.ds(c * (_R // 2), _R // 2)],
            out_sem.at[k])

    for k in range(_NBUF):
        in_copy(k, k).start()

    w1 = w1_ref[...]
    b1 = b1_ref[...]
    w2a = w2a_ref[...]
    w2b = w2b_ref[...]
    b2 = b2_ref[...]

    def group(i, carry):
        for k in range(_NBUF):
            c = i * _NBUF + k
            in_copy(c, k).wait()

            @pl.when(i >= 1)
            def _():
                out_copy(c - _NBUF, k).wait()

            xe = xbufs[k][pl.Slice(0, _R // 2, 2), :]
            xo = xbufs[k][pl.Slice(1, _R // 2, 2), :]
            he = jnp.maximum(jax.lax.dot_general(
                xe, w1,
                dimension_numbers=(((1,), (1,)), ((), ())),
                preferred_element_type=jnp.float32) + b1, 0.0)
            ho = jnp.maximum(jax.lax.dot_general(
                xo, w1,
                dimension_numbers=(((1,), (1,)), ((), ())),
                preferred_element_type=jnp.float32) + b1, 0.0)
            obufs[k][...] = (
                jax.lax.dot_general(
                    he, w2a,
                    dimension_numbers=(((1,), (0,)), ((), ())),
                    preferred_element_type=jnp.float32)
                + jax.lax.dot_general(
                    ho, w2b,
                    dimension_numbers=(((1,), (0,)), ((), ())),
                    preferred_element_type=jnp.float32)
                + b2)

            out_copy(c, k).start()

            @pl.when(c + _NBUF < _S)
            def _():
                in_copy(c + _NBUF, k).start()
        return carry

    jax.lax.fori_loop(0, _GROUPS, group, 0)

    for k in range(_NBUF):
        out_copy(_S - _NBUF + k, k).wait()


def kernel(nodeblocks, x, W_in, b_in, W_out, b_out):
    b1 = b_in.reshape(1, _H)
    w2a = jnp.zeros((_H, 2 * _C), jnp.float32).at[:, :_C].set(W_out.T)
    w2b = jnp.zeros((_H, 2 * _C), jnp.float32).at[:, _C:].set(W_out.T)
    b2 = jnp.concatenate([b_out, b_out]).reshape(1, 2 * _C)
    scratch = (
        [pltpu.VMEM((_R, _F), jnp.float32) for _ in range(_NBUF)]
        + [pltpu.VMEM((_R // 2, 2 * _C), jnp.float32) for _ in range(_NBUF)]
        + [pltpu.SemaphoreType.DMA((_NBUF,)),
           pltpu.SemaphoreType.DMA((_NBUF,))]
    )
    out2 = pl.pallas_call(
        _mlp_kernel,
        in_specs=[
            pl.BlockSpec(memory_space=pltpu.MemorySpace.HBM),
            pl.BlockSpec(memory_space=pltpu.MemorySpace.VMEM),
            pl.BlockSpec(memory_space=pltpu.MemorySpace.VMEM),
            pl.BlockSpec(memory_space=pltpu.MemorySpace.VMEM),
            pl.BlockSpec(memory_space=pltpu.MemorySpace.VMEM),
            pl.BlockSpec(memory_space=pltpu.MemorySpace.VMEM),
        ],
        out_specs=pl.BlockSpec(memory_space=pltpu.MemorySpace.HBM),
        out_shape=jax.ShapeDtypeStruct((_N // 2, 2 * _C), jnp.float32),
        scratch_shapes=scratch,
    )(x, W_in, b1, w2a, w2b, b2)
    return out2.reshape(_N, _C)


# speedup vs baseline: 1.6633x; 1.1656x over previous
import jax
import jax.numpy as jnp
from jax.experimental import pallas as pl
from jax.experimental.pallas import tpu as pltpu

_N, _F, _H, _C = 100000, 128, 128, 64
_R = 2000
_S = _N // _R


def _mlp_kernel(x_hbm, w1_ref, b1_ref, w2_ref, b2_ref, out_hbm):
    w1 = w1_ref[...]
    b1 = b1_ref[...]
    w2 = w2_ref[...]
    b2 = b2_ref[...]

    def inner(x_blk, o_blk):
        xe = x_blk[pl.Slice(0, _R // 2, 2), :]
        xo = x_blk[pl.Slice(1, _R // 2, 2), :]
        he = jnp.maximum(jax.lax.dot_general(
            xe, w1, dimension_numbers=(((1,), (1,)), ((), ())),
            preferred_element_type=jnp.float32) + b1, 0.0)
        ho = jnp.maximum(jax.lax.dot_general(
            xo, w1, dimension_numbers=(((1,), (1,)), ((), ())),
            preferred_element_type=jnp.float32) + b1, 0.0)
        o_blk[:, 0, :] = jax.lax.dot_general(
            he, w2, dimension_numbers=(((1,), (1,)), ((), ())),
            preferred_element_type=jnp.float32) + b2
        o_blk[:, 1, :] = jax.lax.dot_general(
            ho, w2, dimension_numbers=(((1,), (1,)), ((), ())),
            preferred_element_type=jnp.float32) + b2

    pipeline = pltpu.emit_pipeline(
        inner,
        grid=(_S,),
        in_specs=[pl.BlockSpec((_R, _F), lambda c: (c, 0))],
        out_specs=[pl.BlockSpec((_R // 2, 2, _C), lambda c: (c, 0, 0))],
    )
    pipeline(x_hbm, out_hbm.reshape(_N // 2, 2, _C))


def kernel(nodeblocks, x, W_in, b_in, W_out, b_out):
    b1 = b_in.reshape(1, _H)
    b2 = b_out.reshape(1, _C)
    return pl.pallas_call(
        _mlp_kernel,
        in_specs=[
            pl.BlockSpec(memory_space=pltpu.MemorySpace.HBM),
            pl.BlockSpec(memory_space=pltpu.MemorySpace.VMEM),
            pl.BlockSpec(memory_space=pltpu.MemorySpace.VMEM),
            pl.BlockSpec(memory_space=pltpu.MemorySpace.VMEM),
            pl.BlockSpec(memory_space=pltpu.MemorySpace.VMEM),
        ],
        out_specs=pl.BlockSpec(memory_space=pltpu.MemorySpace.HBM),
        out_shape=jax.ShapeDtypeStruct((_N, _C), jnp.float32),
    )(x, W_in, b1, W_out, b2)
